# baseline (speedup 1.0000x reference)
import jax
import jax.numpy as jnp
from jax.experimental import pallas as pl

N = 10000


def _add_kernel(a_ref, b_ref, o_ref):
    o_ref[...] = a_ref[...] + b_ref[...]


def _gcn(x, edge_index, W, b):
    src = edge_index[0]
    dst = edge_index[1]
    loop = jnp.arange(N, dtype=src.dtype)
    src2 = jnp.concatenate([src, loop])
    dst2 = jnp.concatenate([dst, loop])
    ew = jnp.ones(src2.shape[0], dtype=x.dtype)
    deg = jax.ops.segment_sum(ew, dst2, num_segments=N)
    dinv = jnp.where(deg > 0, 1.0 / jnp.sqrt(deg), 0.0)
    norm = dinv[src2] * dinv[dst2]
    xw = x @ W.T
    msg = xw[src2] * norm[:, None]
    out = jax.ops.segment_sum(msg, dst2, num_segments=N)
    return out + b


def _cg(x, edge_index, edge_attr, Wf, bf, Ws, bs):
    src = edge_index[0]
    dst = edge_index[1]
    z = jnp.concatenate([x[dst], x[src], edge_attr], axis=-1)
    m = jax.nn.sigmoid(z @ Wf.T + bf) * jax.nn.softplus(z @ Ws.T + bs)
    summed = jax.ops.segment_sum(m, dst, num_segments=N)
    cnt = jax.ops.segment_sum(jnp.ones((src.shape[0],), dtype=x.dtype), dst, num_segments=N)
    mean = summed / jnp.maximum(cnt, 1.0)[:, None]
    return mean + x


def kernel(x, edge_index, edge_attr, W1, b1, Wf1, bf1, Ws1, bs1, Wf2, bf2, Ws2, bs2, W2, b2):
    h = jax.nn.relu(_gcn(x, edge_index, W1, b1))
    h = jax.nn.relu(_cg(h, edge_index, edge_attr, Wf1, bf1, Ws1, bs1))
    h = jax.nn.relu(_cg(h, edge_index, edge_attr, Wf2, bf2, Ws2, bs2))
    out = _gcn(h, edge_index, W2, b2)
    zero = jnp.zeros_like(out)
    return pl.pallas_call(
        _add_kernel,
        out_shape=jax.ShapeDtypeStruct(out.shape, out.dtype),
    )(out, zero)


# speedup vs baseline: 3.5972x; 3.5972x over previous
import functools

import jax
import jax.numpy as jnp
from jax import lax
from jax.experimental import pallas as pl
from jax.experimental.pallas import tpu as pltpu
from jax.experimental.pallas import tpu_sc as plsc

N_NODE = 10000
N_PAD = 10240
N_EDGE = 320000
HID = 128

NC = 2
NS = 16
NW = NC * NS
EPT = N_EDGE // NW
CH = 80
NCHUNK = EPT // CH
RPT = N_PAD // NS

@functools.cache
def _mesh():
    return plsc.VectorSubcoreMesh(
        core_axis_name="c", subcore_axis_name="s", num_cores=NC, num_subcores=NS
    )

BN = 1024
BE = 2000


def _zero_rows(zb_v, nrows, ncolgrp):
    @pl.loop(0, nrows)
    def _(r):
        @pl.loop(0, ncolgrp)
        def _(j):
            zb_v[r, pl.ds(j * 16, 16)] = jnp.zeros((16,), jnp.float32)


def _count_body(dst_hbm, ones_hbm, out_hbm, idx_v, ones_v, zb_v, acc):
    cid = lax.axis_index("c")
    sid = lax.axis_index("s")
    wid = cid * NS + sid

    pltpu.sync_copy(ones_hbm, ones_v)
    _zero_rows(zb_v, 64, 8)

    @pl.loop(0, RPT // 64)
    def _(k):
        pltpu.sync_copy(zb_v, acc.at[pl.ds(sid * RPT + k * 64, 64)])

    plsc.subcore_barrier()
    ebase = wid * EPT

    @pl.loop(0, NCHUNK)
    def _(c):
        pltpu.sync_copy(dst_hbm.at[pl.ds(ebase + c * CH, CH)], idx_v)
        pltpu.sync_copy(ones_v, acc.at[idx_v], add=True)

    plsc.subcore_barrier()
    pltpu.sync_copy(acc.at[pl.ds(sid * RPT, RPT)],
                    out_hbm.at[cid, pl.ds(sid * RPT, RPT)])


def _sc_count(dst):
    f = pl.kernel(
        _count_body,
        out_type=jax.ShapeDtypeStruct((NC, N_PAD, HID), jnp.float32),
        mesh=_mesh(),
        scratch_types=[
            pltpu.VMEM((CH,), jnp.int32),
            pltpu.VMEM((CH, HID), jnp.float32),
            pltpu.VMEM((64, HID), jnp.float32),
            pltpu.VMEM_SHARED((N_PAD, HID), jnp.float32),
        ],
    )
    return f(dst, jnp.ones((CH, HID), jnp.float32))


def _gather_body(idx_hbm, tab_hbm, out_hbm, idx_v, rows_v):
    wid = lax.axis_index("c") * NS + lax.axis_index("s")
    ebase = wid * EPT

    @pl.loop(0, NCHUNK)
    def _(c):
        eb = ebase + c * CH
        pltpu.sync_copy(idx_hbm.at[pl.ds(eb, CH)], idx_v)
        pltpu.sync_copy(tab_hbm.at[idx_v], rows_v)
        pltpu.sync_copy(rows_v, out_hbm.at[pl.ds(eb, CH)])


def _sc_gather(idx, tab):
    width = tab.shape[1]
    f = pl.kernel(
        _gather_body,
        out_type=jax.ShapeDtypeStruct((N_EDGE, width), jnp.float32),
        mesh=_mesh(),
        scratch_types=[
            pltpu.VMEM((CH,), jnp.int32),
            pltpu.VMEM((CH, width), jnp.float32),
        ],
    )
    return f(idx, tab)


def _scatter_body(idx_hbm, val_hbm, out_hbm, idx_v, val_v, zb_v, acc):
    cid = lax.axis_index("c")
    sid = lax.axis_index("s")
    wid = cid * NS + sid

    _zero_rows(zb_v, 64, 8)

    @pl.loop(0, RPT // 64)
    def _(k):
        pltpu.sync_copy(zb_v, acc.at[pl.ds(sid * RPT + k * 64, 64)])

    plsc.subcore_barrier()
    ebase = wid * EPT

    @pl.loop(0, NCHUNK)
    def _(c):
        eb = ebase + c * CH
        pltpu.sync_copy(idx_hbm.at[pl.ds(eb, CH)], idx_v)
        pltpu.sync_copy(val_hbm.at[pl.ds(eb, CH)], val_v)
        pltpu.sync_copy(val_v, acc.at[idx_v], add=True)

    plsc.subcore_barrier()
    pltpu.sync_copy(acc.at[pl.ds(sid * RPT, RPT)],
                    out_hbm.at[cid, pl.ds(sid * RPT, RPT)])


def _sc_scatter(idx, vals):
    f = pl.kernel(
        _scatter_body,
        out_type=jax.ShapeDtypeStruct((NC, N_PAD, HID), jnp.float32),
        mesh=_mesh(),
        scratch_types=[
            pltpu.VMEM((CH,), jnp.int32),
            pltpu.VMEM((CH, HID), jnp.float32),
            pltpu.VMEM((64, HID), jnp.float32),
            pltpu.VMEM_SHARED((N_PAD, HID), jnp.float32),
        ],
    )
    return f(idx, vals)


def _deg_body(p_ref, dinv_ref, recip_ref):
    p = p_ref[...]
    cnt = (p[0] + p[1])[:, 0:1]
    dinv = lax.rsqrt(cnt + 1.0)
    recip = 1.0 / jnp.maximum(cnt, 1.0)
    dinv_ref[...] = jnp.broadcast_to(dinv, dinv_ref.shape)
    recip_ref[...] = jnp.broadcast_to(recip, recip_ref.shape)


def _tc_deg(parts):
    grid = (N_PAD // BN,)
    out = jax.ShapeDtypeStruct((N_PAD, HID), jnp.float32)
    return pl.pallas_call(
        _deg_body,
        grid=grid,
        in_specs=[pl.BlockSpec((NC, BN, HID), lambda i: (0, i, 0))],
        out_specs=[pl.BlockSpec((BN, HID), lambda i: (i, 0))] * 2,
        out_shape=[out, out],
    )(parts)


def _mm_scale_body(x_ref, w_ref, s_ref, o_ref):
    xw = jnp.dot(x_ref[...], w_ref[...], preferred_element_type=jnp.float32)
    o_ref[...] = xw * s_ref[...]


def _tc_mm_scale(x, w, s):
    grid = (N_PAD // BN,)
    return pl.pallas_call(
        _mm_scale_body,
        grid=grid,
        in_specs=[
            pl.BlockSpec((BN, x.shape[1]), lambda i: (i, 0)),
            pl.BlockSpec(w.shape, lambda i: (0, 0)),
            pl.BlockSpec((BN, HID), lambda i: (i, 0)),
        ],
        out_specs=pl.BlockSpec((BN, w.shape[1]), lambda i: (i, 0)),
        out_shape=jax.ShapeDtypeStruct((N_PAD, w.shape[1]), jnp.float32),
    )(x, w, s)


def _gcn_comb_body(p_ref, xs_ref, dinv_ref, b_ref, o_ref):
    p = p_ref[...]
    s = p[0] + p[1] + xs_ref[...]
    o_ref[...] = jnp.maximum(dinv_ref[...] * s + b_ref[...], 0.0)


def _tc_gcn_combine(parts, xs, dinv_b, bias):
    grid = (N_PAD // BN,)
    return pl.pallas_call(
        _gcn_comb_body,
        grid=grid,
        in_specs=[
            pl.BlockSpec((NC, BN, HID), lambda i: (0, i, 0)),
            pl.BlockSpec((BN, HID), lambda i: (i, 0)),
            pl.BlockSpec((BN, HID), lambda i: (i, 0)),
            pl.BlockSpec((1, HID), lambda i: (0, 0)),
        ],
        out_specs=pl.BlockSpec((BN, HID), lambda i: (i, 0)),
        out_shape=jax.ShapeDtypeStruct((N_PAD, HID), jnp.float32),
    )(parts, xs, dinv_b, bias)


def _tabs_body(h_ref, wd_ref, ws_ref, bd_ref, dt_ref, st_ref):
    h = h_ref[...]
    dt_ref[...] = jnp.dot(h, wd_ref[...], preferred_element_type=jnp.float32) + bd_ref[...]
    st_ref[...] = jnp.dot(h, ws_ref[...], preferred_element_type=jnp.float32)


def _tc_tabs(h, wdT, wsT, bd):
    grid = (N_PAD // BN,)
    out = jax.ShapeDtypeStruct((N_PAD, 2 * HID), jnp.float32)
    return pl.pallas_call(
        _tabs_body,
        grid=grid,
        in_specs=[
            pl.BlockSpec((BN, HID), lambda i: (i, 0)),
            pl.BlockSpec((HID, 2 * HID), lambda i: (0, 0)),
            pl.BlockSpec((HID, 2 * HID), lambda i: (0, 0)),
            pl.BlockSpec((1, 2 * HID), lambda i: (0, 0)),
        ],
        out_specs=[pl.BlockSpec((BN, 2 * HID), lambda i: (i, 0))] * 2,
        out_shape=[out, out],
    )(h, wdT, wsT, bd)


def _edge_body(gd_ref, gs_ref, ea_ref, we_ref, m_ref):
    ec = jnp.dot(ea_ref[...], we_ref[...], preferred_element_type=jnp.float32)
    p = gd_ref[...] + gs_ref[...] + ec
    af = p[:, :HID]
    a2 = p[:, HID:]
    sig = 1.0 / (1.0 + jnp.exp(-af))
    sp = jnp.maximum(a2, 0.0) + jnp.log1p(jnp.exp(-jnp.abs(a2)))
    m_ref[...] = sig * sp


def _tc_edge(gd, gs, ea, weT):
    grid = (N_EDGE // BE,)
    return pl.pallas_call(
        _edge_body,
        grid=grid,
        in_specs=[
            pl.BlockSpec((BE, 2 * HID), lambda i: (i, 0)),
            pl.BlockSpec((BE, 2 * HID), lambda i: (i, 0)),
            pl.BlockSpec((BE, 4), lambda i: (i, 0)),
            pl.BlockSpec((4, 2 * HID), lambda i: (0, 0)),
        ],
        out_specs=pl.BlockSpec((BE, HID), lambda i: (i, 0)),
        out_shape=jax.ShapeDtypeStruct((N_EDGE, HID), jnp.float32),
    )(gd, gs, ea, weT)


def _cg_comb_body(p_ref, h_ref, recip_ref, dinv_ref, o_ref, hs_ref):
    p = p_ref[...]
    mean = (p[0] + p[1]) * recip_ref[...]
    hn = jnp.maximum(mean + h_ref[...], 0.0)
    o_ref[...] = hn
    hs_ref[...] = hn * dinv_ref[...]


def _tc_cg_combine(parts, h, recip_b, dinv_b):
    grid = (N_PAD // BN,)
    out = jax.ShapeDtypeStruct((N_PAD, HID), jnp.float32)
    return pl.pallas_call(
        _cg_comb_body,
        grid=grid,
        in_specs=[
            pl.BlockSpec((NC, BN, HID), lambda i: (0, i, 0)),
            pl.BlockSpec((BN, HID), lambda i: (i, 0)),
            pl.BlockSpec((BN, HID), lambda i: (i, 0)),
            pl.BlockSpec((BN, HID), lambda i: (i, 0)),
        ],
        out_specs=[pl.BlockSpec((BN, HID), lambda i: (i, 0))] * 2,
        out_shape=[out, out],
    )(parts, h, recip_b, dinv_b)


def _final_body(p_ref, hs_ref, dinv_ref, w_ref, b_ref, o_ref):
    p = p_ref[...]
    t = dinv_ref[...] * (p[0] + p[1] + hs_ref[...])
    o_ref[...] = jnp.dot(t, w_ref[...], preferred_element_type=jnp.float32) + b_ref[...]


def _tc_final(parts, hs, dinv_b, w2T, b2p):
    grid = (N_PAD // BN,)
    return pl.pallas_call(
        _final_body,
        grid=grid,
        in_specs=[
            pl.BlockSpec((NC, BN, HID), lambda i: (0, i, 0)),
            pl.BlockSpec((BN, HID), lambda i: (i, 0)),
            pl.BlockSpec((BN, HID), lambda i: (i, 0)),
            pl.BlockSpec((HID, HID), lambda i: (0, 0)),
            pl.BlockSpec((1, HID), lambda i: (0, 0)),
        ],
        out_specs=pl.BlockSpec((BN, HID), lambda i: (i, 0)),
        out_shape=jax.ShapeDtypeStruct((N_PAD, HID), jnp.float32),
    )(parts, hs, dinv_b, w2T, b2p)


def kernel(x, edge_index, edge_attr, W1, b1, Wf1, bf1, Ws1, bs1, Wf2, bf2,
           Ws2, bs2, W2, b2):
    ei = edge_index.astype(jnp.int32)
    src = ei[0]
    dst = ei[1]
    xp = jnp.zeros((N_PAD, x.shape[1]), jnp.float32).at[:N_NODE].set(x)

    cnt_parts = _sc_count(dst)
    dinv_b, recip_b = _tc_deg(cnt_parts)

    xs = _tc_mm_scale(xp, W1.T, dinv_b)
    g1 = _sc_gather(src, xs)
    p1 = _sc_scatter(dst, g1)
    h = _tc_gcn_combine(p1, xs, dinv_b, b1.reshape(1, HID))

    hs = None
    for Wf, bf, Ws, bs in ((Wf1, bf1, Ws1, bs1), (Wf2, bf2, Ws2, bs2)):
        wdT = jnp.concatenate([Wf[:, :HID], Ws[:, :HID]], axis=0).T
        wsT = jnp.concatenate([Wf[:, HID:2 * HID], Ws[:, HID:2 * HID]], axis=0).T
        weT = jnp.concatenate([Wf[:, 2 * HID:], Ws[:, 2 * HID:]], axis=0).T
        bd = jnp.concatenate([bf, bs]).reshape(1, 2 * HID)
        dt, st = _tc_tabs(h, wdT, wsT, bd)
        gd = _sc_gather(dst, dt)
        gs = _sc_gather(src, st)
        m = _tc_edge(gd, gs, edge_attr, weT)
        pm = _sc_scatter(dst, m)
        h, hs = _tc_cg_combine(pm, h, recip_b, dinv_b)

    g2 = _sc_gather(src, hs)
    p2 = _sc_scatter(dst, g2)
    w2T = jnp.zeros((HID, HID), jnp.float32).at[:, :2].set(W2.T)
    b2p = jnp.zeros((1, HID), jnp.float32).at[0, :2].set(b2)
    out = _tc_final(p2, hs, dinv_b, w2T, b2p)
    return out[:N_NODE, :2]
